# initial kernel scaffold (unmeasured)
import jax
import jax.numpy as jnp
from jax import lax
from jax.experimental import pallas as pl
from jax.experimental.pallas import tpu as pltpu

N_DEV = 16


def kernel(x, w_mat, scale_x, scale_w):
    m_per, k = x.shape
    k_w, n_per = w_mat.shape

    def body(x_ref, w_ref, sx_ref, sw_ref, out_ref, xg_ref, send_sems, recv_sems):
        my = lax.axis_index("i")
        left = lax.rem(my + N_DEV - 1, N_DEV)
        right = lax.rem(my + 1, N_DEV)

        barrier_sem = pltpu.get_barrier_semaphore()
        for nbr in (left, right):
            pl.semaphore_signal(
                barrier_sem, inc=1,
                device_id=(nbr,), device_id_type=pl.DeviceIdType.MESH,
            )
        pl.semaphore_wait(barrier_sem, 2)

        xg_ref[pl.ds(my, 1)] = x_ref[...][None]

        for h in range(N_DEV - 1):
            slot = lax.rem(my + N_DEV - h, N_DEV)
            rdma = pltpu.make_async_remote_copy(
                src_ref=xg_ref.at[slot],
                dst_ref=xg_ref.at[slot],
                send_sem=send_sems.at[h],
                recv_sem=recv_sems.at[h],
                device_id=(right,),
                device_id_type=pl.DeviceIdType.MESH,
            )
            rdma.start()
            rdma.wait()

        xf = xg_ref[...].reshape(N_DEV * m_per, k).astype(jnp.bfloat16)
        wf = w_ref[...].astype(jnp.bfloat16)
        acc = jnp.dot(xf, wf, preferred_element_type=jnp.float32)
        out_ref[...] = acc * (sx_ref[0] * sw_ref[0])

    return pl.pallas_call(
        body,
        out_shape=jax.ShapeDtypeStruct((N_DEV * m_per, n_per), jnp.float32),
        in_specs=[
            pl.BlockSpec(memory_space=pltpu.VMEM),
            pl.BlockSpec(memory_space=pltpu.VMEM),
            pl.BlockSpec(memory_space=pltpu.SMEM),
            pl.BlockSpec(memory_space=pltpu.SMEM),
        ],
        out_specs=pl.BlockSpec(memory_space=pltpu.VMEM),
        scratch_shapes=[
            pltpu.VMEM((N_DEV, m_per, k), x.dtype),
            pltpu.SemaphoreType.DMA((N_DEV - 1,)),
            pltpu.SemaphoreType.DMA((N_DEV - 1,)),
        ],
        compiler_params=pltpu.CompilerParams(collective_id=0),
    )(x, w_mat, scale_x, scale_w)


# baseline (device time: 226411 ns/iter reference)
import jax
import jax.numpy as jnp
from jax import lax
from jax.experimental import pallas as pl
from jax.experimental.pallas import tpu as pltpu

N_DEV = 16


def kernel(x, w_mat, scale_x, scale_w):
    m_per, k = x.shape
    k_w, n_per = w_mat.shape

    def body(x_ref, w_ref, sx_ref, sw_ref, out_ref, xg_ref, send_sems, recv_sems):
        my = lax.axis_index("i")
        left = lax.rem(my + N_DEV - 1, N_DEV)
        right = lax.rem(my + 1, N_DEV)

        barrier_sem = pltpu.get_barrier_semaphore()
        for nbr in (left, right):
            pl.semaphore_signal(
                barrier_sem, inc=1,
                device_id=(nbr,), device_id_type=pl.DeviceIdType.MESH,
            )
        pl.semaphore_wait(barrier_sem, 2)

        xg_ref[pl.ds(my, 1)] = x_ref[...].astype(jnp.float8_e4m3fn)[None]

        for h in range(N_DEV - 1):
            slot = lax.rem(my + N_DEV - h, N_DEV)
            rdma = pltpu.make_async_remote_copy(
                src_ref=xg_ref.at[slot],
                dst_ref=xg_ref.at[slot],
                send_sem=send_sems.at[h],
                recv_sem=recv_sems.at[h],
                device_id=(right,),
                device_id_type=pl.DeviceIdType.MESH,
            )
            rdma.start()
            rdma.wait()

        wf = w_ref[...].astype(jnp.bfloat16)
        scale = sx_ref[0] * sw_ref[0]
        for s in range(N_DEV):
            acc = jnp.dot(
                xg_ref[s].astype(jnp.bfloat16), wf,
                preferred_element_type=jnp.float32,
            )
            out_ref[pl.ds(s * m_per, m_per), :] = acc * scale

    return pl.pallas_call(
        body,
        out_shape=jax.ShapeDtypeStruct((N_DEV * m_per, n_per), jnp.float32),
        in_specs=[
            pl.BlockSpec(memory_space=pltpu.VMEM),
            pl.BlockSpec(memory_space=pltpu.VMEM),
            pl.BlockSpec(memory_space=pltpu.SMEM),
            pl.BlockSpec(memory_space=pltpu.SMEM),
        ],
        out_specs=pl.BlockSpec(memory_space=pltpu.VMEM),
        scratch_shapes=[
            pltpu.VMEM((N_DEV, m_per, k), jnp.float8_e4m3fn),
            pltpu.SemaphoreType.DMA((N_DEV - 1,)),
            pltpu.SemaphoreType.DMA((N_DEV - 1,)),
        ],
        compiler_params=pltpu.CompilerParams(collective_id=0),
    )(x, w_mat, scale_x, scale_w)


# device time: 119286 ns/iter; 1.8981x vs baseline; 1.8981x over previous
import jax
import jax.numpy as jnp
from jax import lax
from jax.experimental import pallas as pl
from jax.experimental.pallas import tpu as pltpu

N_DEV = 16
H_CW = 8
H_CCW = N_DEV - 1 - H_CW


def kernel(x, w_mat, scale_x, scale_w):
    m_per, k = x.shape
    k_w, n_per = w_mat.shape

    def body(x_ref, w_ref, sx_ref, sw_ref, out_ref, xg_ref,
             cw_send_sems, cw_recv_sems, ccw_send_sems, ccw_recv_sems):
        my = lax.axis_index("i")
        left = lax.rem(my + N_DEV - 1, N_DEV)
        right = lax.rem(my + 1, N_DEV)

        def slot(off):
            return lax.rem(my + N_DEV + off, N_DEV)

        def fwd(direction_slot, h, cw):
            return pltpu.make_async_remote_copy(
                src_ref=xg_ref.at[direction_slot],
                dst_ref=xg_ref.at[direction_slot],
                send_sem=(cw_send_sems if cw else ccw_send_sems).at[h],
                recv_sem=(cw_recv_sems if cw else ccw_recv_sems).at[h],
                device_id=(right if cw else left,),
                device_id_type=pl.DeviceIdType.MESH,
            )

        barrier_sem = pltpu.get_barrier_semaphore()
        for nbr in (left, right):
            pl.semaphore_signal(
                barrier_sem, inc=1,
                device_id=(nbr,), device_id_type=pl.DeviceIdType.MESH,
            )
        pl.semaphore_wait(barrier_sem, 2)

        xg_ref[pl.ds(my, 1)] = x_ref[...].astype(jnp.float8_e4m3fn)[None]
        pending = []
        d = fwd(slot(0), 0, cw=True)
        d.start()
        pending.append(d)
        d = fwd(slot(0), 0, cw=False)
        d.start()
        pending.append(d)

        wf = w_ref[...].astype(jnp.bfloat16)
        scale = sx_ref[0] * sw_ref[0]

        def compute_slab(s):
            acc = jnp.dot(
                xg_ref[s].astype(jnp.bfloat16), wf,
                preferred_element_type=jnp.float32,
            )
            out_ref[pl.ds(s * m_per, m_per), :] = acc * scale

        compute_slab(slot(0))

        for h in range(H_CW):
            recv = fwd(slot(-h - 1), h, cw=True)
            recv.wait_recv()
            if h + 1 < H_CW:
                d = fwd(slot(-h - 1), h + 1, cw=True)
                d.start()
                pending.append(d)
            if h < H_CCW:
                recv_c = fwd(slot(h + 1), h, cw=False)
                recv_c.wait_recv()
                if h + 1 < H_CCW:
                    d = fwd(slot(h + 1), h + 1, cw=False)
                    d.start()
                    pending.append(d)
            compute_slab(slot(-h - 1))
            if h < H_CCW:
                compute_slab(slot(h + 1))

        for d in pending:
            d.wait_send()

    return pl.pallas_call(
        body,
        out_shape=jax.ShapeDtypeStruct((N_DEV * m_per, n_per), jnp.float32),
        in_specs=[
            pl.BlockSpec(memory_space=pltpu.VMEM),
            pl.BlockSpec(memory_space=pltpu.VMEM),
            pl.BlockSpec(memory_space=pltpu.SMEM),
            pl.BlockSpec(memory_space=pltpu.SMEM),
        ],
        out_specs=pl.BlockSpec(memory_space=pltpu.VMEM),
        scratch_shapes=[
            pltpu.VMEM((N_DEV, m_per, k), jnp.float8_e4m3fn),
            pltpu.SemaphoreType.DMA((H_CW,)),
            pltpu.SemaphoreType.DMA((H_CW,)),
            pltpu.SemaphoreType.DMA((H_CCW,)),
            pltpu.SemaphoreType.DMA((H_CCW,)),
        ],
        compiler_params=pltpu.CompilerParams(collective_id=0),
    )(x, w_mat, scale_x, scale_w)


# device time: 101732 ns/iter; 2.2256x vs baseline; 1.1726x over previous
import jax
import jax.numpy as jnp
from jax import lax
from jax.experimental import pallas as pl
from jax.experimental.pallas import tpu as pltpu

N_DEV = 16
H = 8
N_HALF = 2


def kernel(x, w_mat, scale_x, scale_w):
    m_per, k = x.shape
    k_w, n_per = w_mat.shape
    m_half = m_per // N_HALF

    def body(x_ref, w_ref, sx_ref, sw_ref, out_ref, xg_ref,
             cw_send, cw_recv, ccw_send, ccw_recv):
        my = lax.axis_index("i")
        left = lax.rem(my + N_DEV - 1, N_DEV)
        right = lax.rem(my + 1, N_DEV)

        def slot(off):
            return lax.rem(my + N_DEV + off, N_DEV)

        def fwd(s, h, q, cw):
            return pltpu.make_async_remote_copy(
                src_ref=xg_ref.at[s, q],
                dst_ref=xg_ref.at[s, q],
                send_sem=(cw_send if cw else ccw_send).at[h, q],
                recv_sem=(cw_recv if cw else ccw_recv).at[h, q],
                device_id=(right if cw else left,),
                device_id_type=pl.DeviceIdType.MESH,
            )

        barrier_sem = pltpu.get_barrier_semaphore()
        for nbr in (left, right):
            pl.semaphore_signal(
                barrier_sem, inc=1,
                device_id=(nbr,), device_id_type=pl.DeviceIdType.MESH,
            )
        pl.semaphore_wait(barrier_sem, 2)

        xg_ref[pl.ds(my, 1)] = (
            x_ref[...].astype(jnp.float8_e4m3fn)
            .reshape(1, N_HALF, m_half, k)
        )
        pending = []

        def start(s, h, q, cw):
            d = fwd(s, h, q, cw)
            d.start()
            pending.append(d)

        for q in range(N_HALF):
            start(slot(0), 0, q, cw=True)
            start(slot(0), 0, q, cw=False)

        wf = w_ref[...].astype(jnp.bfloat16)
        scale = sx_ref[0] * sw_ref[0]

        def compute_slab(s):
            acc = jnp.dot(
                xg_ref[s].reshape(m_per, k).astype(jnp.bfloat16), wf,
                preferred_element_type=jnp.float32,
            )
            out_ref[pl.ds(s * m_per, m_per), :] = acc * scale

        compute_slab(slot(0))

        for h in range(H):
            if h < H - 1:
                recv = fwd(slot(-h - 1), h, 0, cw=True)
                recv.wait_recv()
                start(slot(-h - 1), h + 1, 0, cw=True)
                recv = fwd(slot(h + 1), h, 1, cw=False)
                recv.wait_recv()
                start(slot(h + 1), h + 1, 1, cw=False)
                recv = fwd(slot(-h - 1), h, 1, cw=True)
                recv.wait_recv()
                if h + 1 < H - 1:
                    start(slot(-h - 1), h + 1, 1, cw=True)
                recv = fwd(slot(h + 1), h, 0, cw=False)
                recv.wait_recv()
                if h + 1 < H - 1:
                    start(slot(h + 1), h + 1, 0, cw=False)
                compute_slab(slot(-h - 1))
                compute_slab(slot(h + 1))
            else:
                recv = fwd(slot(H), h, 0, cw=True)
                recv.wait_recv()
                recv = fwd(slot(H), h, 1, cw=False)
                recv.wait_recv()
                compute_slab(slot(H))

        for d in pending:
            d.wait_send()

    return pl.pallas_call(
        body,
        out_shape=jax.ShapeDtypeStruct((N_DEV * m_per, n_per), jnp.float32),
        in_specs=[
            pl.BlockSpec(memory_space=pltpu.VMEM),
            pl.BlockSpec(memory_space=pltpu.VMEM),
            pl.BlockSpec(memory_space=pltpu.SMEM),
            pl.BlockSpec(memory_space=pltpu.SMEM),
        ],
        out_specs=pl.BlockSpec(memory_space=pltpu.VMEM),
        scratch_shapes=[
            pltpu.VMEM((N_DEV, N_HALF, m_half, k), jnp.float8_e4m3fn),
            pltpu.SemaphoreType.DMA((H, N_HALF)),
            pltpu.SemaphoreType.DMA((H, N_HALF)),
            pltpu.SemaphoreType.DMA((H, N_HALF)),
            pltpu.SemaphoreType.DMA((H, N_HALF)),
        ],
        compiler_params=pltpu.CompilerParams(collective_id=0),
    )(x, w_mat, scale_x, scale_w)
